# baseline (device time: 123771 ns/iter reference)
import jax
import jax.numpy as jnp
from jax import lax
from jax.experimental import pallas as pl
from jax.experimental.pallas import tpu as pltpu

N_DEV = 16


def kernel(x, w_mat):
    m_per, k = x.shape
    _, n_per = w_mat.shape

    def body(x_ref, w_ref, out_ref, comm_ref, w_bf_ref, send_sems, recv_sems):
        my_pos = lax.axis_index("i")
        left = (my_pos - 1) % N_DEV
        right = (my_pos + 1) % N_DEV

        barrier_sem = pltpu.get_barrier_semaphore()
        for nbr in [left, right]:
            pl.semaphore_signal(
                barrier_sem, inc=1,
                device_id=(nbr,), device_id_type=pl.DeviceIdType.MESH,
            )
        pl.semaphore_wait(barrier_sem, 2)

        w_bf_ref[...] = w_ref[...].astype(jnp.bfloat16)
        comm_ref[0] = x_ref[...].astype(jnp.bfloat16)

        def gemm_store(origin, chunk):
            acc = jnp.dot(chunk, w_bf_ref[...], preferred_element_type=jnp.float32)
            out_ref[pl.ds(origin * m_per, m_per), :] = acc * (
                1.0 / (1.0 + jnp.exp(-acc))
            )

        gemm_store(my_pos, comm_ref[0])

        for h in range(N_DEV - 1):
            send_slot = h % 2
            recv_slot = (h + 1) % 2
            rdma = pltpu.make_async_remote_copy(
                src_ref=comm_ref.at[send_slot],
                dst_ref=comm_ref.at[recv_slot],
                send_sem=send_sems.at[send_slot],
                recv_sem=recv_sems.at[recv_slot],
                device_id=(right,),
                device_id_type=pl.DeviceIdType.MESH,
            )
            rdma.start()
            rdma.wait()

            origin = (my_pos - h - 1) % N_DEV
            gemm_store(origin, comm_ref[recv_slot])

    return pl.pallas_call(
        body,
        out_shape=jax.ShapeDtypeStruct((N_DEV * m_per, n_per), jnp.float32),
        in_specs=[
            pl.BlockSpec(memory_space=pltpu.VMEM),
            pl.BlockSpec(memory_space=pltpu.VMEM),
        ],
        out_specs=pl.BlockSpec(memory_space=pltpu.VMEM),
        scratch_shapes=[
            pltpu.VMEM((2, m_per, k), jnp.bfloat16),
            pltpu.VMEM((k, n_per), jnp.bfloat16),
            pltpu.SemaphoreType.DMA((2,)),
            pltpu.SemaphoreType.DMA((2,)),
        ],
        compiler_params=pltpu.CompilerParams(collective_id=0),
    )(x, w_mat)


# device time: 82708 ns/iter; 1.4965x vs baseline; 1.4965x over previous
import jax
import jax.numpy as jnp
from jax import lax
from jax.experimental import pallas as pl
from jax.experimental.pallas import tpu as pltpu

N_DEV = 16
H_R = 8
H_L = 7


def kernel(x, w_mat):
    m_per, k = x.shape
    _, n_per = w_mat.shape

    def body(
        x_ref, w_ref, out_ref,
        comm_r, comm_l, w_bf_ref,
        send_r, recv_r, send_l, recv_l,
    ):
        my_pos = lax.axis_index("i")
        left = (my_pos - 1) % N_DEV
        right = (my_pos + 1) % N_DEV

        barrier_sem = pltpu.get_barrier_semaphore()
        for nbr in [left, right]:
            pl.semaphore_signal(
                barrier_sem, inc=1,
                device_id=(nbr,), device_id_type=pl.DeviceIdType.MESH,
            )
        pl.semaphore_wait(barrier_sem, 2)

        w_bf_ref[...] = w_ref[...].astype(jnp.bfloat16)
        x_bf = x_ref[...].astype(jnp.bfloat16)
        comm_r[0] = x_bf
        comm_l[0] = x_bf

        def gemm_store(origin, chunk):
            acc = jnp.dot(chunk, w_bf_ref[...], preferred_element_type=jnp.float32)
            out_ref[pl.ds(origin * m_per, m_per), :] = acc * (
                1.0 / (1.0 + jnp.exp(-acc))
            )

        def hop(comm, send_sems, recv_sems, h, target):
            rdma = pltpu.make_async_remote_copy(
                src_ref=comm.at[h % 2],
                dst_ref=comm.at[(h + 1) % 2],
                send_sem=send_sems.at[h % 2],
                recv_sem=recv_sems.at[(h + 1) % 2],
                device_id=(target,),
                device_id_type=pl.DeviceIdType.MESH,
            )
            rdma.start()
            return rdma

        gemm_store(my_pos, comm_r[0])

        for h in range(H_R):
            rr = hop(comm_r, send_r, recv_r, h, right)
            if h < H_L:
                rl = hop(comm_l, send_l, recv_l, h, left)
            rr.wait()
            gemm_store((my_pos - h - 1) % N_DEV, comm_r[(h + 1) % 2])
            if h < H_L:
                rl.wait()
                gemm_store((my_pos + h + 1) % N_DEV, comm_l[(h + 1) % 2])

    return pl.pallas_call(
        body,
        out_shape=jax.ShapeDtypeStruct((N_DEV * m_per, n_per), jnp.float32),
        in_specs=[
            pl.BlockSpec(memory_space=pltpu.VMEM),
            pl.BlockSpec(memory_space=pltpu.VMEM),
        ],
        out_specs=pl.BlockSpec(memory_space=pltpu.VMEM),
        scratch_shapes=[
            pltpu.VMEM((2, m_per, k), jnp.bfloat16),
            pltpu.VMEM((2, m_per, k), jnp.bfloat16),
            pltpu.VMEM((k, n_per), jnp.bfloat16),
            pltpu.SemaphoreType.DMA((2,)),
            pltpu.SemaphoreType.DMA((2,)),
            pltpu.SemaphoreType.DMA((2,)),
            pltpu.SemaphoreType.DMA((2,)),
        ],
        compiler_params=pltpu.CompilerParams(collective_id=0),
    )(x, w_mat)


# device time: 67735 ns/iter; 1.8273x vs baseline; 1.2211x over previous
import jax
import jax.numpy as jnp
from jax import lax
from jax.experimental import pallas as pl
from jax.experimental.pallas import tpu as pltpu

N_DEV = 16
H_R = 8
H_L = 7
B = 3


def kernel(x, w_mat):
    m_per, k = x.shape
    _, n_per = w_mat.shape

    def body(
        x_ref, w_ref, out_ref,
        x_bf, comm_r, comm_l, w_bf_ref,
        send_r, recv_r, send_l, recv_l,
    ):
        my_pos = lax.axis_index("i")
        left = (my_pos - 1) % N_DEV
        right = (my_pos + 1) % N_DEV

        barrier_sem = pltpu.get_barrier_semaphore()
        for nbr in [left, right]:
            pl.semaphore_signal(
                barrier_sem, inc=1,
                device_id=(nbr,), device_id_type=pl.DeviceIdType.MESH,
            )
        pl.semaphore_wait(barrier_sem, 2)

        w_bf_ref[...] = w_ref[...].astype(jnp.bfloat16)
        x_bf[...] = x_ref[...].astype(jnp.bfloat16)

        def gemm_store(origin, chunk):
            acc = jnp.dot(chunk, w_bf_ref[...], preferred_element_type=jnp.float32)
            out_ref[pl.ds(origin * m_per, m_per), :] = acc * (
                1.0 / (1.0 + jnp.exp(-acc))
            )

        def rdma_hop(comm, send_sems, recv_sems, h, target):
            src = x_bf if h == 0 else comm.at[(h - 1) % B]
            return pltpu.make_async_remote_copy(
                src_ref=src,
                dst_ref=comm.at[h % B],
                send_sem=send_sems.at[h % B],
                recv_sem=recv_sems.at[h % B],
                device_id=(target,),
                device_id_type=pl.DeviceIdType.MESH,
            )

        rdma_hop(comm_r, send_r, recv_r, 0, right).start()
        rdma_hop(comm_l, send_l, recv_l, 0, left).start()
        gemm_store(my_pos, x_bf[...])

        for h in range(H_R):
            rr = rdma_hop(comm_r, send_r, recv_r, h, right)
            rr.wait_recv()
            rr.wait_send()
            if h + 1 < H_R:
                rdma_hop(comm_r, send_r, recv_r, h + 1, right).start()
            if h < H_L:
                rl = rdma_hop(comm_l, send_l, recv_l, h, left)
                rl.wait_recv()
                rl.wait_send()
                if h + 1 < H_L:
                    rdma_hop(comm_l, send_l, recv_l, h + 1, left).start()
            gemm_store((my_pos - h - 1) % N_DEV, comm_r[h % B])
            if h < H_L:
                gemm_store((my_pos + h + 1) % N_DEV, comm_l[h % B])

    return pl.pallas_call(
        body,
        out_shape=jax.ShapeDtypeStruct((N_DEV * m_per, n_per), jnp.float32),
        in_specs=[
            pl.BlockSpec(memory_space=pltpu.VMEM),
            pl.BlockSpec(memory_space=pltpu.VMEM),
        ],
        out_specs=pl.BlockSpec(memory_space=pltpu.VMEM),
        scratch_shapes=[
            pltpu.VMEM((m_per, k), jnp.bfloat16),
            pltpu.VMEM((B, m_per, k), jnp.bfloat16),
            pltpu.VMEM((B, m_per, k), jnp.bfloat16),
            pltpu.VMEM((k, n_per), jnp.bfloat16),
            pltpu.SemaphoreType.DMA((B,)),
            pltpu.SemaphoreType.DMA((B,)),
            pltpu.SemaphoreType.DMA((B,)),
            pltpu.SemaphoreType.DMA((B,)),
        ],
        compiler_params=pltpu.CompilerParams(collective_id=0),
    )(x, w_mat)


# device time: 56916 ns/iter; 2.1746x vs baseline; 1.1901x over previous
import jax
import jax.numpy as jnp
from jax import lax
from jax.experimental import pallas as pl
from jax.experimental.pallas import tpu as pltpu

N_DEV = 16
H_R = 8
H_L = 7
B = 3
S = 4


def kernel(x, w_mat):
    m_per, k = x.shape
    _, n_per = w_mat.shape

    def body(
        x_ref, w_ref, out_ref,
        x_bf, comm_r, comm_l, w_bf_ref,
        send_r, recv_r, send_l, recv_l,
    ):
        my_pos = lax.axis_index("i")
        left = (my_pos - 1) % N_DEV
        right = (my_pos + 1) % N_DEV

        barrier_sem = pltpu.get_barrier_semaphore()
        for nbr in [left, right]:
            pl.semaphore_signal(
                barrier_sem, inc=1,
                device_id=(nbr,), device_id_type=pl.DeviceIdType.MESH,
            )
        pl.semaphore_wait(barrier_sem, 2)

        w_bf_ref[...] = w_ref[...].astype(jnp.bfloat16)
        x_bf[...] = x_ref[...].astype(jnp.bfloat16)

        def gemm_store(origin, chunk):
            acc = jnp.dot(chunk, w_bf_ref[...], preferred_element_type=jnp.float32)
            out_ref[pl.ds(origin * m_per, m_per), :] = acc * (
                1.0 / (1.0 + jnp.exp(-acc))
            )

        sub_m = m_per // S

        def rdma_hop(comm, send_sems, recv_sems, h, s, target):
            rows = pl.ds(s * sub_m, sub_m)
            src = x_bf.at[rows] if h == 0 else comm.at[(h - 1) % B, rows]
            return pltpu.make_async_remote_copy(
                src_ref=src,
                dst_ref=comm.at[h % B, rows],
                send_sem=send_sems.at[h % B, s],
                recv_sem=recv_sems.at[h % B, s],
                device_id=(target,),
                device_id_type=pl.DeviceIdType.MESH,
            )

        for s in range(S):
            rdma_hop(comm_r, send_r, recv_r, 0, s, right).start()
            rdma_hop(comm_l, send_l, recv_l, 0, s, left).start()
        gemm_store(my_pos, x_bf[...])

        for h in range(H_R):
            for s in range(S):
                rr = rdma_hop(comm_r, send_r, recv_r, h, s, right)
                rr.wait_recv()
                rr.wait_send()
                if h + 1 < H_R:
                    rdma_hop(comm_r, send_r, recv_r, h + 1, s, right).start()
                if h < H_L:
                    rl = rdma_hop(comm_l, send_l, recv_l, h, s, left)
                    rl.wait_recv()
                    rl.wait_send()
                    if h + 1 < H_L:
                        rdma_hop(comm_l, send_l, recv_l, h + 1, s, left).start()
            gemm_store((my_pos - h - 1) % N_DEV, comm_r[h % B])
            if h < H_L:
                gemm_store((my_pos + h + 1) % N_DEV, comm_l[h % B])

    return pl.pallas_call(
        body,
        out_shape=jax.ShapeDtypeStruct((N_DEV * m_per, n_per), jnp.float32),
        in_specs=[
            pl.BlockSpec(memory_space=pltpu.VMEM),
            pl.BlockSpec(memory_space=pltpu.VMEM),
        ],
        out_specs=pl.BlockSpec(memory_space=pltpu.VMEM),
        scratch_shapes=[
            pltpu.VMEM((m_per, k), jnp.bfloat16),
            pltpu.VMEM((B, m_per, k), jnp.bfloat16),
            pltpu.VMEM((B, m_per, k), jnp.bfloat16),
            pltpu.VMEM((k, n_per), jnp.bfloat16),
            pltpu.SemaphoreType.DMA((B, S)),
            pltpu.SemaphoreType.DMA((B, S)),
            pltpu.SemaphoreType.DMA((B, S)),
            pltpu.SemaphoreType.DMA((B, S)),
        ],
        compiler_params=pltpu.CompilerParams(collective_id=0),
    )(x, w_mat)


# device time: 52050 ns/iter; 2.3779x vs baseline; 1.0935x over previous
import jax
import jax.numpy as jnp
from jax import lax
from jax.experimental import pallas as pl
from jax.experimental.pallas import tpu as pltpu

N_DEV = 16
H = 8
B = 3
S = 4

RING = [0, 1, 5, 9, 13, 14, 10, 6, 2, 3, 7, 11, 15, 12, 8, 4]
INV = [RING.index(p) for p in range(N_DEV)]


def kernel(x, w_mat):
    m_per, k = x.shape
    _, n_per = w_mat.shape

    def body(
        x_ref, w_ref, out_ref,
        x_bf, comm_r, comm_l, w_bf_ref,
        send_r, recv_r, send_l, recv_l,
    ):
        my_pos = lax.axis_index("i")

        def table(idx, vals):
            out = jnp.int32(vals[0])
            for j in range(1, len(vals)):
                out = jnp.where(idx == j, jnp.int32(vals[j]), out)
            return out

        ridx = table(my_pos, INV)
        right = table((ridx + 1) % N_DEV, RING)
        left = table((ridx + N_DEV - 1) % N_DEV, RING)

        def origin_r(h):
            return table((ridx + N_DEV - h - 1) % N_DEV, RING)

        def origin_l(h):
            return table((ridx + h + 1) % N_DEV, RING)

        w_bf_ref[...] = w_ref[...].astype(jnp.bfloat16)
        x_bf[...] = x_ref[...].astype(jnp.bfloat16)

        barrier_sem = pltpu.get_barrier_semaphore()
        for nbr in [left, right]:
            pl.semaphore_signal(
                barrier_sem, inc=1,
                device_id=(nbr,), device_id_type=pl.DeviceIdType.MESH,
            )
        pl.semaphore_wait(barrier_sem, 2)

        def gemm_store(origin, chunk, row_off, nrows):
            acc = jnp.dot(chunk, w_bf_ref[...], preferred_element_type=jnp.float32)
            out_ref[pl.ds(origin * m_per + row_off, nrows), :] = acc * (
                1.0 / (1.0 + jnp.exp(-acc))
            )

        sub_m = m_per // S

        def rdma_hop(comm, send_sems, recv_sems, h, s, target):
            rows = pl.ds(s * sub_m, sub_m)
            src = x_bf.at[rows] if h == 0 else comm.at[(h - 1) % B, rows]
            return pltpu.make_async_remote_copy(
                src_ref=src,
                dst_ref=comm.at[h % B, rows],
                send_sem=send_sems.at[h % B, s],
                recv_sem=recv_sems.at[h % B, s],
                device_id=(target,),
                device_id_type=pl.DeviceIdType.MESH,
            )

        def subs_r(h):
            return range(S) if h < H - 1 else (0, 1)

        def subs_l(h):
            return range(S) if h < H - 1 else (2, 3)

        for s in range(S):
            rdma_hop(comm_r, send_r, recv_r, 0, s, right).start()
            rdma_hop(comm_l, send_l, recv_l, 0, s, left).start()
        gemm_store(my_pos, x_bf[...], 0, m_per)

        for h in range(H):
            for s in range(S):
                if s in subs_r(h):
                    rr = rdma_hop(comm_r, send_r, recv_r, h, s, right)
                    rr.wait_recv()
                    rr.wait_send()
                    if h + 1 < H and s in subs_r(h + 1):
                        rdma_hop(comm_r, send_r, recv_r, h + 1, s, right).start()
                if s in subs_l(h):
                    rl = rdma_hop(comm_l, send_l, recv_l, h, s, left)
                    rl.wait_recv()
                    rl.wait_send()
                    if h + 1 < H and s in subs_l(h + 1):
                        rdma_hop(comm_l, send_l, recv_l, h + 1, s, left).start()
            if h < H - 1:
                gemm_store(origin_r(h), comm_r[h % B], 0, m_per)
                gemm_store(origin_l(h), comm_l[h % B], 0, m_per)
            else:
                half = m_per // 2
                gemm_store(origin_r(h), comm_r[h % B, :half], 0, half)
                gemm_store(origin_l(h), comm_l[h % B, half:], half, half)

    return pl.pallas_call(
        body,
        out_shape=jax.ShapeDtypeStruct((N_DEV * m_per, n_per), jnp.float32),
        in_specs=[
            pl.BlockSpec(memory_space=pltpu.VMEM),
            pl.BlockSpec(memory_space=pltpu.VMEM),
        ],
        out_specs=pl.BlockSpec(memory_space=pltpu.VMEM),
        scratch_shapes=[
            pltpu.VMEM((m_per, k), jnp.bfloat16),
            pltpu.VMEM((B, m_per, k), jnp.bfloat16),
            pltpu.VMEM((B, m_per, k), jnp.bfloat16),
            pltpu.VMEM((k, n_per), jnp.bfloat16),
            pltpu.SemaphoreType.DMA((B, S)),
            pltpu.SemaphoreType.DMA((B, S)),
            pltpu.SemaphoreType.DMA((B, S)),
            pltpu.SemaphoreType.DMA((B, S)),
        ],
        compiler_params=pltpu.CompilerParams(collective_id=0),
    )(x, w_mat)


# device time: 51918 ns/iter; 2.3840x vs baseline; 1.0025x over previous
import jax
import jax.numpy as jnp
from jax import lax
from jax.experimental import pallas as pl
from jax.experimental.pallas import tpu as pltpu

N_DEV = 16
H = 8
B = 3
S = 4

RING = [0, 1, 5, 9, 13, 14, 10, 6, 2, 3, 7, 11, 15, 12, 8, 4]
INV = [RING.index(p) for p in range(N_DEV)]


def kernel(x, w_mat):
    m_per, k = x.shape
    _, n_per = w_mat.shape

    def body(
        x_ref, w_ref, out_ref,
        x_bf, comm_r, comm_l, w_bf_ref,
        send_r, recv_r, send_l, recv_l,
    ):
        my_pos = lax.axis_index("i")

        def table(idx, vals):
            out = jnp.int32(vals[0])
            for j in range(1, len(vals)):
                out = jnp.where(idx == j, jnp.int32(vals[j]), out)
            return out

        ridx = table(my_pos, INV)
        right = table((ridx + 1) % N_DEV, RING)
        left = table((ridx + N_DEV - 1) % N_DEV, RING)

        def origin_r(h):
            return table((ridx + N_DEV - h - 1) % N_DEV, RING)

        def origin_l(h):
            return table((ridx + h + 1) % N_DEV, RING)

        barrier_sem = pltpu.get_barrier_semaphore()
        for nbr in [left, right]:
            pl.semaphore_signal(
                barrier_sem, inc=1,
                device_id=(nbr,), device_id_type=pl.DeviceIdType.MESH,
            )
        x_bf[...] = x_ref[...].astype(jnp.bfloat16)
        pl.semaphore_wait(barrier_sem, 2)

        def gemm_store(origin, chunk, row_off, nrows):
            acc = jnp.dot(chunk, w_bf_ref[...], preferred_element_type=jnp.float32)
            out_ref[pl.ds(origin * m_per + row_off, nrows), :] = acc * (
                1.0 / (1.0 + jnp.exp(-acc))
            )

        sub_m = m_per // S

        def rdma_hop(comm, send_sems, recv_sems, h, s, target):
            rows = pl.ds(s * sub_m, sub_m)
            src = x_bf.at[rows] if h == 0 else comm.at[(h - 1) % B, rows]
            return pltpu.make_async_remote_copy(
                src_ref=src,
                dst_ref=comm.at[h % B, rows],
                send_sem=send_sems.at[h % B, s],
                recv_sem=recv_sems.at[h % B, s],
                device_id=(target,),
                device_id_type=pl.DeviceIdType.MESH,
            )

        def subs_r(h):
            return range(S) if h < H - 1 else (0, 1)

        def subs_l(h):
            return range(S) if h < H - 1 else (2, 3)

        for s in range(S):
            rdma_hop(comm_r, send_r, recv_r, 0, s, right).start()
            rdma_hop(comm_l, send_l, recv_l, 0, s, left).start()
        w_bf_ref[...] = w_ref[...].astype(jnp.bfloat16)
        gemm_store(my_pos, x_bf[...], 0, m_per)

        for h in range(H):
            for s in range(S):
                if s in subs_r(h):
                    rr = rdma_hop(comm_r, send_r, recv_r, h, s, right)
                    rr.wait_recv()
                    rr.wait_send()
                    if h + 1 < H and s in subs_r(h + 1):
                        rdma_hop(comm_r, send_r, recv_r, h + 1, s, right).start()
                if s in subs_l(h):
                    rl = rdma_hop(comm_l, send_l, recv_l, h, s, left)
                    rl.wait_recv()
                    rl.wait_send()
                    if h + 1 < H and s in subs_l(h + 1):
                        rdma_hop(comm_l, send_l, recv_l, h + 1, s, left).start()
            if h < H - 1:
                gemm_store(origin_r(h), comm_r[h % B], 0, m_per)
                gemm_store(origin_l(h), comm_l[h % B], 0, m_per)
            else:
                half = m_per // 2
                gemm_store(origin_r(h), comm_r[h % B, :half], 0, half)
                gemm_store(origin_l(h), comm_l[h % B, half:], half, half)

    return pl.pallas_call(
        body,
        out_shape=jax.ShapeDtypeStruct((N_DEV * m_per, n_per), jnp.float32),
        in_specs=[
            pl.BlockSpec(memory_space=pltpu.VMEM),
            pl.BlockSpec(memory_space=pltpu.VMEM),
        ],
        out_specs=pl.BlockSpec(memory_space=pltpu.VMEM),
        scratch_shapes=[
            pltpu.VMEM((m_per, k), jnp.bfloat16),
            pltpu.VMEM((B, m_per, k), jnp.bfloat16),
            pltpu.VMEM((B, m_per, k), jnp.bfloat16),
            pltpu.VMEM((k, n_per), jnp.bfloat16),
            pltpu.SemaphoreType.DMA((B, S)),
            pltpu.SemaphoreType.DMA((B, S)),
            pltpu.SemaphoreType.DMA((B, S)),
            pltpu.SemaphoreType.DMA((B, S)),
        ],
        compiler_params=pltpu.CompilerParams(collective_id=0),
    )(x, w_mat)
